# baseline (speedup 1.0000x reference)
import functools

import jax
import jax.numpy as jnp
from jax import lax
from jax.experimental import pallas as pl
from jax.experimental.pallas import tpu as pltpu
from jax.experimental.pallas import tpu_sc as plsc

_VOCAB = 1000000
_D = 64
_BATCH = 4096
_HIST = 200

_NC, _NS = 2, 16
_NW = _NC * _NS
_B = _BATCH * _HIST
_BPW = _B // _NW
_CH = 256
_NG = _BPW // _CH
_NT = (_NG - 1) // 3

_VC = 8192
_NB = -(-_VOCAB // _VC)

_mesh = plsc.VectorSubcoreMesh(core_axis_name="c", subcore_axis_name="s")


def _tp_body(in_ref, out_ref):
    blk = in_ref[...]
    out_ref[...] = jnp.concatenate(
        [blk.T, jnp.zeros((_VC, 128 - _D), jnp.float32)], axis=1
    )


def _tc_transpose_pad(wt):
    return pl.pallas_call(
        _tp_body,
        grid=(_NB,),
        in_specs=[pl.BlockSpec((_D, _VC), lambda i: (0, i))],
        out_specs=pl.BlockSpec((_VC, 128), lambda i: (i, 0)),
        out_shape=jax.ShapeDtypeStruct((_VOCAB, 128), jnp.float32),
    )(wt)


@functools.partial(
    pl.kernel,
    mesh=_mesh,
    out_type=jax.ShapeDtypeStruct((_B, 128), jnp.float32),
    scratch_types=[
        pltpu.VMEM((_BPW,), jnp.int32),
        pltpu.VMEM((3, _CH, 128), jnp.float32),
        pltpu.SemaphoreType.DMA,
        pltpu.SemaphoreType.DMA,
        pltpu.SemaphoreType.DMA,
        pltpu.SemaphoreType.DMA,
        pltpu.SemaphoreType.DMA,
        pltpu.SemaphoreType.DMA,
    ],
    compiler_params=pltpu.CompilerParams(use_tc_tiling_on_sc=True),
)
def _sc_gather(idx_hbm, table_hbm, out_hbm, idx_v, rows_v, g0, g1, g2, s0, s1, s2):
    wid = lax.axis_index("s") * _NC + lax.axis_index("c")
    base = wid * _BPW
    pltpu.sync_copy(idx_hbm.at[pl.ds(base, _BPW)], idx_v)

    gsem = (g0, g1, g2)
    ssem = (s0, s1, s2)

    def fire_ga(g, b):
        pltpu.async_copy(
            table_hbm.at[idx_v.at[pl.ds(g * _CH, _CH)]], rows_v.at[b], gsem[b]
        )

    def drain_ga(g, b):
        pltpu.make_async_copy(
            table_hbm.at[idx_v.at[pl.ds(g * _CH, _CH)]], rows_v.at[b], gsem[b]
        ).wait()

    def fire_st(g, b):
        pltpu.async_copy(
            rows_v.at[b], out_hbm.at[pl.ds(base + g * _CH, _CH)], ssem[b]
        )

    def drain_st(g, b):
        pltpu.make_async_copy(
            rows_v.at[b], out_hbm.at[pl.ds(base + g * _CH, _CH)], ssem[b]
        ).wait()

    fire_ga(0, 0)
    fire_ga(1, 1)
    fire_ga(2, 2)
    drain_ga(0, 0)
    fire_st(0, 0)

    def body(h, carry):
        g = 3 * h + 1
        drain_st(g - 1, 0)
        fire_ga(g + 2, 0)
        drain_ga(g, 1)
        fire_st(g, 1)
        drain_st(g, 1)
        fire_ga(g + 3, 1)
        drain_ga(g + 1, 2)
        fire_st(g + 1, 2)
        drain_st(g + 1, 2)
        fire_ga(g + 4, 2)
        drain_ga(g + 2, 0)
        fire_st(g + 2, 0)
        return carry

    lax.fori_loop(0, _NT - 1, body, 0)

    g = 3 * (_NT - 1) + 1
    drain_st(g - 1, 0)
    fire_ga(g + 2, 0)
    drain_ga(g, 1)
    fire_st(g, 1)
    drain_st(g, 1)
    drain_ga(g + 1, 2)
    fire_st(g + 1, 2)
    drain_st(g + 1, 2)
    drain_ga(g + 2, 0)
    fire_st(g + 2, 0)
    drain_st(g + 2, 0)


def kernel(token_ids, embedding_weight):
    idx = token_ids.astype(jnp.int32).reshape(_B)
    tbl = _tc_transpose_pad(embedding_weight.T)
    out = _sc_gather(idx, tbl)
    return out[:, :_D].reshape(_BATCH, _HIST, _D)


# speedup vs baseline: 1.0248x; 1.0248x over previous
import functools

import jax
import jax.numpy as jnp
from jax import lax
from jax.experimental import pallas as pl
from jax.experimental.pallas import tpu as pltpu
from jax.experimental.pallas import tpu_sc as plsc

_VOCAB = 1000000
_D = 64
_BATCH = 4096
_HIST = 200

_NC, _NS = 2, 16
_NW = _NC * _NS
_B = _BATCH * _HIST
_BPW = _B // _NW
_CH = 256
_NG = _BPW // _CH

_VC = 16384
_NB = -(-_VOCAB // _VC)

_mesh = plsc.VectorSubcoreMesh(core_axis_name="c", subcore_axis_name="s")


def _tp_body(in_ref, out_ref):
    blk = in_ref[...]
    out_ref[...] = jnp.concatenate(
        [blk.T, jnp.zeros((_VC, 128 - _D), jnp.float32)], axis=1
    )


def _tc_transpose_pad(wt):
    return pl.pallas_call(
        _tp_body,
        grid=(_NB,),
        in_specs=[pl.BlockSpec((_D, _VC), lambda i: (0, i))],
        out_specs=pl.BlockSpec((_VC, 128), lambda i: (i, 0)),
        out_shape=jax.ShapeDtypeStruct((_VOCAB, 128), jnp.float32),
    )(wt)


@functools.partial(
    pl.kernel,
    mesh=_mesh,
    out_type=jax.ShapeDtypeStruct((_B, 128), jnp.float32),
    scratch_types=[
        pltpu.VMEM((_BPW,), jnp.int32),
        pltpu.VMEM((2, _CH, 128), jnp.float32),
        pltpu.SemaphoreType.DMA,
        pltpu.SemaphoreType.DMA,
        pltpu.SemaphoreType.DMA,
        pltpu.SemaphoreType.DMA,
    ],
    compiler_params=pltpu.CompilerParams(use_tc_tiling_on_sc=True),
)
def _sc_gather(idx_hbm, table_hbm, out_hbm, idx_v, rows_v, g0, g1, s0, s1):
    wid = lax.axis_index("s") * _NC + lax.axis_index("c")
    base = wid * _BPW
    pltpu.sync_copy(idx_hbm.at[pl.ds(base, _BPW)], idx_v)

    gsem = (g0, g1)
    ssem = (s0, s1)

    def fire_ga(g, b):
        pltpu.async_copy(
            table_hbm.at[idx_v.at[pl.ds(g * _CH, _CH)]], rows_v.at[b], gsem[b]
        )

    def drain_ga(g, b):
        pltpu.make_async_copy(
            table_hbm.at[idx_v.at[pl.ds(g * _CH, _CH)]], rows_v.at[b], gsem[b]
        ).wait()

    def fire_st(g, b):
        pltpu.async_copy(
            rows_v.at[b], out_hbm.at[pl.ds(base + g * _CH, _CH)], ssem[b]
        )

    def drain_st(g, b):
        pltpu.make_async_copy(
            rows_v.at[b], out_hbm.at[pl.ds(base + g * _CH, _CH)], ssem[b]
        ).wait()

    fire_ga(0, 0)
    fire_ga(1, 1)
    drain_ga(0, 0)
    fire_st(0, 0)
    drain_st(0, 0)
    fire_ga(2, 0)
    drain_ga(1, 1)
    fire_st(1, 1)

    def body(h, carry):
        ge = 2 * h
        drain_st(ge - 1, 1)
        fire_ga(ge + 1, 1)
        drain_ga(ge, 0)
        fire_st(ge, 0)
        drain_st(ge, 0)
        fire_ga(ge + 2, 0)
        drain_ga(ge + 1, 1)
        fire_st(ge + 1, 1)
        return carry

    lax.fori_loop(1, _NG // 2 - 1, body, 0)

    ge = _NG - 2
    drain_st(ge - 1, 1)
    fire_ga(ge + 1, 1)
    drain_ga(ge, 0)
    fire_st(ge, 0)
    drain_st(ge, 0)
    drain_ga(ge + 1, 1)
    fire_st(ge + 1, 1)
    drain_st(ge + 1, 1)


def kernel(token_ids, embedding_weight):
    idx = token_ids.astype(jnp.int32).reshape(_B)
    tbl = _tc_transpose_pad(embedding_weight.T)
    out = _sc_gather(idx, tbl)
    return out[:, :_D].reshape(_BATCH, _HIST, _D)


# speedup vs baseline: 1.0339x; 1.0089x over previous
import functools

import jax
import jax.numpy as jnp
from jax import lax
from jax.experimental import pallas as pl
from jax.experimental.pallas import tpu as pltpu
from jax.experimental.pallas import tpu_sc as plsc

_VOCAB = 1000000
_D = 64
_BATCH = 4096
_HIST = 200

_NC, _NS = 2, 16
_NW = _NC * _NS
_B = _BATCH * _HIST
_BPW = _B // _NW
_CH = 256
_NG = _BPW // _CH

_VC = 32768
_NB = -(-_VOCAB // _VC)

_mesh = plsc.VectorSubcoreMesh(core_axis_name="c", subcore_axis_name="s")


def _tp_body(in_ref, out_ref):
    blk = in_ref[...]
    out_ref[...] = jnp.concatenate(
        [blk.T, jnp.zeros((_VC, 128 - _D), jnp.float32)], axis=1
    )


def _tc_transpose_pad(wt):
    return pl.pallas_call(
        _tp_body,
        grid=(_NB,),
        in_specs=[pl.BlockSpec((_D, _VC), lambda i: (0, i))],
        out_specs=pl.BlockSpec((_VC, 128), lambda i: (i, 0)),
        out_shape=jax.ShapeDtypeStruct((_VOCAB, 128), jnp.float32),
    )(wt)


@functools.partial(
    pl.kernel,
    mesh=_mesh,
    out_type=jax.ShapeDtypeStruct((_B, 128), jnp.float32),
    scratch_types=[
        pltpu.VMEM((_BPW,), jnp.int32),
        pltpu.VMEM((2, _CH, 128), jnp.float32),
        pltpu.SemaphoreType.DMA,
        pltpu.SemaphoreType.DMA,
        pltpu.SemaphoreType.DMA,
        pltpu.SemaphoreType.DMA,
    ],
    compiler_params=pltpu.CompilerParams(use_tc_tiling_on_sc=True),
)
def _sc_gather(idx_hbm, table_hbm, out_hbm, idx_v, rows_v, g0, g1, s0, s1):
    wid = lax.axis_index("s") * _NC + lax.axis_index("c")
    base = wid * _BPW
    pltpu.sync_copy(idx_hbm.at[pl.ds(base, _BPW)], idx_v)

    gsem = (g0, g1)
    ssem = (s0, s1)

    def fire_ga(g, b):
        pltpu.async_copy(
            table_hbm.at[idx_v.at[pl.ds(g * _CH, _CH)]], rows_v.at[b], gsem[b]
        )

    def drain_ga(g, b):
        pltpu.make_async_copy(
            table_hbm.at[idx_v.at[pl.ds(g * _CH, _CH)]], rows_v.at[b], gsem[b]
        ).wait()

    def fire_st(g, b):
        pltpu.async_copy(
            rows_v.at[b], out_hbm.at[pl.ds(base + g * _CH, _CH)], ssem[b]
        )

    def drain_st(g, b):
        pltpu.make_async_copy(
            rows_v.at[b], out_hbm.at[pl.ds(base + g * _CH, _CH)], ssem[b]
        ).wait()

    fire_ga(0, 0)
    fire_ga(1, 1)
    drain_ga(0, 0)
    fire_st(0, 0)
    drain_st(0, 0)
    fire_ga(2, 0)
    drain_ga(1, 1)
    fire_st(1, 1)

    def body(h, carry):
        ge = 2 * h
        drain_st(ge - 1, 1)
        fire_ga(ge + 1, 1)
        drain_ga(ge, 0)
        fire_st(ge, 0)
        drain_st(ge, 0)
        fire_ga(ge + 2, 0)
        drain_ga(ge + 1, 1)
        fire_st(ge + 1, 1)
        return carry

    lax.fori_loop(1, _NG // 2 - 1, body, 0)

    ge = _NG - 2
    drain_st(ge - 1, 1)
    fire_ga(ge + 1, 1)
    drain_ga(ge, 0)
    fire_st(ge, 0)
    drain_st(ge, 0)
    drain_ga(ge + 1, 1)
    fire_st(ge + 1, 1)
    drain_st(ge + 1, 1)


def kernel(token_ids, embedding_weight):
    idx = token_ids.astype(jnp.int32).reshape(_B)
    tbl = _tc_transpose_pad(embedding_weight.T)
    out = _sc_gather(idx, tbl)
    return out[:, :_D].reshape(_BATCH, _HIST, _D)
